# Initial kernel scaffold; baseline (speedup 1.0000x reference)
import functools

import numpy as np
import jax
import jax.numpy as jnp
from jax import lax
from jax.experimental import pallas as pl
from jax.experimental.pallas import tpu as pltpu
from jax.experimental.pallas import tpu_sc as plsc

B = 16384
T = 12
D = 24
V = 28
RD = T * D
NC = 2
NS = 16
NW = NC * NS
BPW = B // NW
CHUNK = 64
NCHUNK = BPW // CHUNK
NJ = RD // 16

_RJ = np.stack([(16 * j + np.arange(16)) // D for j in range(NJ)]).astype(np.int32)


def _body(x_hbm, w_hbm, p_hbm, out_hbm, x_v, w_v, p_v, c_v, ob_v, sem0, sem1):
    wid = lax.axis_index("s") * NC + lax.axis_index("c")
    xbase = wid * (BPW * T)
    obase = wid * (BPW * RD)

    pltpu.sync_copy(x_hbm.at[pl.ds(xbase, BPW * T)], x_v)
    pltpu.sync_copy(w_hbm, w_v)
    pltpu.sync_copy(p_hbm, p_v)

    def build(i, carry):
        e = i * 16 + lax.iota(jnp.int32, 16)
        q = e // RD
        rp = e - q * RD
        d = rp - (rp // D) * D
        wv = plsc.load_gather(w_v, [q * D + d])
        pv = plsc.load_gather(p_v, [rp])
        c_v[pl.ds(i * 16, 16)] = wv + pv
        return carry

    lax.fori_loop(0, (V * RD) // 16, build, 0)

    sems = (sem0, sem1)
    copies = [None, None]
    for ci in range(NCHUNK):
        buf = ci % 2
        if copies[buf] is not None:
            copies[buf].wait()

        def row(b, carry):
            bx = (ci * CHUNK + b) * T
            bo = b * RD
            for j in range(NJ):
                rj = jnp.asarray(_RJ[j])
                xg = plsc.load_gather(x_v, [bx + rj])
                idx = xg * RD + (16 * j + lax.iota(jnp.int32, 16))
                ob_v[buf, pl.ds(bo + 16 * j, 16)] = plsc.load_gather(c_v, [idx])
            return carry

        lax.fori_loop(0, CHUNK, row, 0)
        copies[buf] = pltpu.async_copy(
            ob_v.at[buf],
            out_hbm.at[pl.ds(obase + ci * CHUNK * RD, CHUNK * RD)],
            sems[buf],
        )
    copies[0].wait()
    copies[1].wait()


_mesh = plsc.VectorSubcoreMesh(core_axis_name="c", subcore_axis_name="s")

_ebd = functools.partial(
    pl.kernel,
    mesh=_mesh,
    out_type=jax.ShapeDtypeStruct((B * T * D,), jnp.float32),
    scratch_types=[
        pltpu.VMEM((BPW * T,), jnp.int32),
        pltpu.VMEM((V * D,), jnp.float32),
        pltpu.VMEM((T * D,), jnp.float32),
        pltpu.VMEM((V * RD,), jnp.float32),
        pltpu.VMEM((2, CHUNK * RD), jnp.float32),
        pltpu.SemaphoreType.DMA,
        pltpu.SemaphoreType.DMA,
    ],
)(_body)


@jax.jit
def kernel(X, word_table, pos_table):
    out = _ebd(X.reshape(-1), word_table.reshape(-1), pos_table.reshape(-1))
    return out.reshape(B, T, D)


# speedup vs baseline: 4.4404x; 4.4404x over previous
import functools

import jax
import jax.numpy as jnp
from jax import lax
from jax.experimental import pallas as pl
from jax.experimental.pallas import tpu as pltpu
from jax.experimental.pallas import tpu_sc as plsc

B = 16384
T = 12
D = 24
V = 28
RD = T * D
NC = 2
NS = 16
NW = NC * NS
BPW = B // NW
CHUNK = 64
NCHUNK = BPW // CHUNK
NJ = RD // 16


def _lane_patterns():
    lane = lax.iota(jnp.int32, 16)
    d0 = lane
    d1 = jnp.where(lane < 8, lane + 16, lane - 8)
    d2 = lane + 8
    return lane, d0, d1, d2


def _token_vec(j, lane):
    m, r = divmod(j, 3)
    if r == 0:
        return lane * 0 + 2 * m
    if r == 1:
        return (lane >= 8).astype(jnp.int32) + 2 * m
    return lane * 0 + (2 * m + 1)


def _body(x_hbm, w_hbm, p_hbm, out_hbm, x_v, w_v, p_v, c_v, ob_v, sem0, sem1):
    wid = lax.axis_index("s") * NC + lax.axis_index("c")
    xbase = wid * (BPW * T)
    obase = wid * (BPW * RD)

    pltpu.sync_copy(x_hbm.at[pl.ds(xbase, BPW * T)], x_v)
    pltpu.sync_copy(w_hbm, w_v)
    pltpu.sync_copy(p_hbm, p_v)

    lane, d0, d1, d2 = _lane_patterns()
    ds_pat = (d0, d1, d2)

    def build(w, carry):
        dd = carry
        for j in range(NJ):
            pv = p_v[pl.ds(16 * j, 16)]
            wv = plsc.load_gather(w_v, [w * D + dd[j % 3]])
            c_v[pl.ds(w * RD + 16 * j, 16)] = wv + pv
        return carry

    lax.fori_loop(0, V, build, ds_pat)

    rjs = tuple(_token_vec(j, lane) for j in range(NJ))

    sems = (sem0, sem1)
    copies = [None, None]
    for ci in range(NCHUNK):
        buf = ci % 2
        if copies[buf] is not None:
            copies[buf].wait()

        def row(b, carry):
            rs = carry
            bx = (ci * CHUNK + b) * T
            bo = b * RD
            for j in range(NJ):
                xg = plsc.load_gather(x_v, [bx + rs[j]])
                idx = xg * RD + (lax.iota(jnp.int32, 16) + 16 * j)
                ob_v[buf, pl.ds(bo + 16 * j, 16)] = plsc.load_gather(c_v, [idx])
            return carry

        lax.fori_loop(0, CHUNK, row, rjs)
        copies[buf] = pltpu.async_copy(
            ob_v.at[buf],
            out_hbm.at[pl.ds(obase + ci * CHUNK * RD, CHUNK * RD)],
            sems[buf],
        )
    copies[0].wait()
    copies[1].wait()


_mesh = plsc.VectorSubcoreMesh(core_axis_name="c", subcore_axis_name="s")

_ebd = functools.partial(
    pl.kernel,
    mesh=_mesh,
    compiler_params=pltpu.CompilerParams(needs_layout_passes=False),
    out_type=jax.ShapeDtypeStruct((B * T * D,), jnp.float32),
    scratch_types=[
        pltpu.VMEM((BPW * T,), jnp.int32),
        pltpu.VMEM((V * D,), jnp.float32),
        pltpu.VMEM((T * D,), jnp.float32),
        pltpu.VMEM((V * RD,), jnp.float32),
        pltpu.VMEM((2, CHUNK * RD), jnp.float32),
        pltpu.SemaphoreType.DMA,
        pltpu.SemaphoreType.DMA,
    ],
)(_body)


@jax.jit
def kernel(X, word_table, pos_table):
    out = _ebd(X.reshape(-1), word_table.reshape(-1), pos_table.reshape(-1))
    return out.reshape(B, T, D)


# speedup vs baseline: 6.8629x; 1.5455x over previous
import functools

import jax
import jax.numpy as jnp
from jax import lax
from jax.experimental import pallas as pl
from jax.experimental.pallas import tpu as pltpu
from jax.experimental.pallas import tpu_sc as plsc

B = 16384
T = 12
D = 24
V = 28
NC = 2
NS = 16
NW = NC * NS
RPW = (B * T) // NW
NV = RPW // 16
CHUNK = 1536
NCHUNK = RPW // CHUNK


def _tpat():
    lane = lax.iota(jnp.int32, 16)
    t0 = jnp.where(lane >= 12, lane - 12, lane)
    t1 = jnp.where(lane >= 8, lane - 8, lane + 4)
    t2 = jnp.where(lane >= 4, lane - 4, lane + 8)
    return t0, t1, t2


def _dpat():
    lane = lax.iota(jnp.int32, 16)
    d0 = lane
    d1 = jnp.where(lane < 8, lane + 16, lane - 8)
    d2 = lane + 8
    return d0, d1, d2


def _body(
    x_hbm, w_hbm, p_hbm, out_hbm, x_v, w_v, p_v, c_v, idx_v, g_v, c_sh, sem0, sem1
):
    cid = lax.axis_index("c")
    sid = lax.axis_index("s")
    wid = sid * NC + cid
    xbase = wid * RPW

    pltpu.sync_copy(x_hbm.at[pl.ds(xbase, RPW)], x_v)

    @pl.when(sid == 0)
    def _build():
        pltpu.sync_copy(w_hbm, w_v)
        pltpu.sync_copy(p_hbm, p_v)
        lane = lax.iota(jnp.int32, 16)
        dpat = _dpat()
        tadd = (lane * 0, (lane >= 8).astype(jnp.int32), lane * 0 + 1)

        def build(w, carry):
            dd, tt = carry
            for j in range(T * D // 16):
                m, r = divmod(j, 3)
                pv = p_v[pl.ds(16 * j, 16)]
                wv = plsc.load_gather(w_v, [w * D + dd[r]])
                plsc.store_scatter(c_v, [w * T + 2 * m + tt[r], dd[r]], wv + pv)
            return carry

        lax.fori_loop(0, V, build, (dpat, tadd))
        pltpu.sync_copy(c_v, c_sh)

    plsc.subcore_barrier()

    tpat = _tpat()

    def mkidx(i, carry):
        tt = carry
        for r in range(3):
            xv = x_v[pl.ds((3 * i + r) * 16, 16)]
            idx_v[pl.ds((3 * i + r) * 16, 16)] = xv * T + tt[r]
        return carry

    lax.fori_loop(0, NV // 3, mkidx, tpat)

    c2 = c_sh
    ocopies = [None, None]
    for k in range(NCHUNK):
        buf = k % 2
        if ocopies[buf] is not None:
            ocopies[buf].wait()
        pltpu.async_copy(
            c2.at[idx_v.at[pl.ds(k * CHUNK, CHUNK)]],
            g_v.at[buf],
            sem0,
        ).wait()
        ocopies[buf] = pltpu.async_copy(
            g_v.at[buf],
            out_hbm.at[pl.ds(xbase + k * CHUNK, CHUNK)],
            sem1,
        )
    ocopies[0].wait()
    ocopies[1].wait()


_mesh = plsc.VectorSubcoreMesh(core_axis_name="c", subcore_axis_name="s")

_ebd = functools.partial(
    pl.kernel,
    mesh=_mesh,
    compiler_params=pltpu.CompilerParams(
        needs_layout_passes=False, use_tc_tiling_on_sc=False
    ),
    out_type=jax.ShapeDtypeStruct((B * T, D), jnp.float32),
    scratch_types=[
        pltpu.VMEM((RPW,), jnp.int32),
        pltpu.VMEM((V * D,), jnp.float32),
        pltpu.VMEM((T * D,), jnp.float32),
        pltpu.VMEM((V * T, D), jnp.float32),
        pltpu.VMEM((RPW,), jnp.int32),
        pltpu.VMEM((2, CHUNK, D), jnp.float32),
        pltpu.VMEM_SHARED((V * T, D), jnp.float32),
        pltpu.SemaphoreType.DMA,
        pltpu.SemaphoreType.DMA,
    ],
)(_body)


@jax.jit
def kernel(X, word_table, pos_table):
    out = _ebd(X.reshape(-1), word_table.reshape(-1), pos_table.reshape(-1))
    return out.reshape(B, T, D)
